# Initial kernel scaffold; baseline (speedup 1.0000x reference)
import jax
import jax.numpy as jnp
from jax.experimental import pallas as pl


def kernel(state, weights):
    raise NotImplementedError("write your pallas kernel here")


# speedup vs baseline: 496.7737x; 496.7737x over previous
import jax
import jax.numpy as jnp
from jax import lax
from jax.experimental import pallas as pl
from jax.experimental.pallas import tpu as pltpu
from jax.experimental.pallas import tpu_sc as plsc

_A = 8
_B = 16384
_T = 32
_NB = 64
_TABLE = _T * _NB * _NB
_HALF = _TABLE // 2
_HT = _T // 2
_NW = 32
_CHUNK = (_A * _B) // _NW
_LANES = 16


def _tile_q_body(s0_hbm, s1_hbm, w_hbm, out_hbm, tbl, s0, s1, acc):
    wid = lax.axis_index("s") * 2 + lax.axis_index("c")
    base = wid * _CHUNK
    act = wid // 4
    pltpu.sync_copy(s0_hbm.at[pl.ds(base, _CHUNK)], s0)
    pltpu.sync_copy(s1_hbm.at[pl.ds(base, _CHUNK)], s1)

    for h in (0, 1):
        pltpu.sync_copy(w_hbm.at[pl.ds(act * _TABLE + h * _HALF, _HALF)], tbl)

        def chunk_body(i, carry, h=h):
            o = i * _LANES
            v0 = s0[pl.ds(o, _LANES)]
            v1 = s1[pl.ds(o, _LANES)]
            a = jnp.zeros((_LANES,), jnp.float32)
            for tl in range(_HT):
                tg = h * _HT + tl
                off = jnp.float32(tg / 2048.0)
                i0 = jnp.clip(((v0 + off) * 64.0).astype(jnp.int32), 0, 63)
                i1 = jnp.clip(((v1 + off) * 64.0).astype(jnp.int32), 0, 63)
                flat = tl * (_NB * _NB) + i0 * _NB + i1
                a = a + plsc.load_gather(tbl, [flat])
            if h == 0:
                acc[pl.ds(o, _LANES)] = a
            else:
                acc[pl.ds(o, _LANES)] = acc[pl.ds(o, _LANES)] + a
            return carry

        lax.fori_loop(0, _CHUNK // _LANES, chunk_body, 0)

    pltpu.sync_copy(acc, out_hbm.at[pl.ds(base, _CHUNK)])


def kernel(state, weights):
    s0 = state[:, :, 0].reshape(-1)
    s1 = state[:, :, 1].reshape(-1)
    w = weights.reshape(-1)
    mesh = plsc.VectorSubcoreMesh(core_axis_name="c", subcore_axis_name="s")
    run = pl.kernel(
        _tile_q_body,
        out_type=jax.ShapeDtypeStruct((_A * _B,), jnp.float32),
        mesh=mesh,
        compiler_params=pltpu.CompilerParams(needs_layout_passes=False),
        scratch_types=[
            pltpu.VMEM((_HALF,), jnp.float32),
            pltpu.VMEM((_CHUNK,), jnp.float32),
            pltpu.VMEM((_CHUNK,), jnp.float32),
            pltpu.VMEM((_CHUNK,), jnp.float32),
        ],
    )
    return run(s0, s1, w)
